# baseline (speedup 1.0000x reference)
import jax
import jax.numpy as jnp
from jax.experimental import pallas as pl


def _add_kernel(we_ref, pos_ref, out_ref):
    out_ref[...] = we_ref[...] + pos_ref[...][None, :, :]


def kernel(word_embeddings, pos_table):
    B, S, D = word_embeddings.shape
    BS = 1024
    grid = (S // BS, B)
    return pl.pallas_call(
        _add_kernel,
        grid=grid,
        in_specs=[
            pl.BlockSpec((1, BS, D), lambda s, b: (b, s, 0)),
            pl.BlockSpec((BS, D), lambda s, b: (s, 0)),
        ],
        out_specs=pl.BlockSpec((1, BS, D), lambda s, b: (b, s, 0)),
        out_shape=jax.ShapeDtypeStruct((B, S, D), word_embeddings.dtype),
    )(word_embeddings, pos_table)


# speedup vs baseline: 1.0432x; 1.0432x over previous
import jax
import jax.numpy as jnp
from jax.experimental import pallas as pl
from jax.experimental.pallas import tpu as pltpu


def _add_kernel(we_ref, pos_ref, out_ref):
    out_ref[...] = we_ref[...] + pos_ref[...][None, :, :]


def kernel(word_embeddings, pos_table):
    B, S, D = word_embeddings.shape
    BS = 2048
    grid = (S // BS, B)
    return pl.pallas_call(
        _add_kernel,
        grid=grid,
        in_specs=[
            pl.BlockSpec((1, BS, D), lambda s, b: (b, s, 0)),
            pl.BlockSpec((BS, D), lambda s, b: (s, 0)),
        ],
        out_specs=pl.BlockSpec((1, BS, D), lambda s, b: (b, s, 0)),
        out_shape=jax.ShapeDtypeStruct((B, S, D), word_embeddings.dtype),
        compiler_params=pltpu.CompilerParams(dimension_semantics=("parallel", "parallel")),
    )(word_embeddings, pos_table)


# speedup vs baseline: 1.0445x; 1.0012x over previous
import jax
import jax.numpy as jnp
from jax.experimental import pallas as pl


def _add_kernel(we_ref, pos_ref, out_ref):
    out_ref[...] = we_ref[...] + pos_ref[...][None, :, :]


def kernel(word_embeddings, pos_table):
    B, S, D = word_embeddings.shape
    BS = 2048
    grid = (S // BS, B)
    return pl.pallas_call(
        _add_kernel,
        grid=grid,
        in_specs=[
            pl.BlockSpec((1, BS, D), lambda s, b: (b, s, 0)),
            pl.BlockSpec((BS, D), lambda s, b: (s, 0)),
        ],
        out_specs=pl.BlockSpec((1, BS, D), lambda s, b: (b, s, 0)),
        out_shape=jax.ShapeDtypeStruct((B, S, D), word_embeddings.dtype),
    )(word_embeddings, pos_table)


# speedup vs baseline: 1.0452x; 1.0007x over previous
import jax
import jax.numpy as jnp
from jax.experimental import pallas as pl
from jax.experimental.pallas import tpu as pltpu

_CHUNK = 1024
_DEPTH = 4


def _row_start(g, B, S):
    p = g // B
    r = g % B
    return r * S + p * _CHUNK


def _make_body(B, S, D, NG, NP):
    def body(we_hbm, pos_hbm, o_hbm, we_buf, pos_buf, out_buf,
             we_sem, pos_sem, out_sem):
        def we_copy(g, slot):
            rs = _row_start(g, B, S)
            return pltpu.make_async_copy(
                we_hbm.at[pl.ds(rs, _CHUNK), :], we_buf.at[slot],
                we_sem.at[slot])

        def pos_copy(p, pslot):
            return pltpu.make_async_copy(
                pos_hbm.at[pl.ds(p * _CHUNK, _CHUNK), :], pos_buf.at[pslot],
                pos_sem.at[pslot])

        def out_copy(g, slot):
            rs = _row_start(g, B, S)
            return pltpu.make_async_copy(
                out_buf.at[slot], o_hbm.at[pl.ds(rs, _CHUNK), :],
                out_sem.at[slot])

        for k in range(_DEPTH):
            we_copy(k, k).start()
        pos_copy(0, 0).start()
        pos_copy(1, 1).start()

        def step(g, _):
            slot = jax.lax.rem(g, _DEPTH)
            p = g // B
            r = jax.lax.rem(g, B)
            pslot = jax.lax.rem(p, 2)

            we_copy(g, slot).wait()

            @pl.when(r == 0)
            def _():
                pos_copy(p, pslot).wait()

            @pl.when(g >= _DEPTH)
            def _():
                out_copy(g - _DEPTH, slot).wait()

            out_buf[slot] = we_buf[slot] + pos_buf[pslot]
            out_copy(g, slot).start()

            @pl.when(g + _DEPTH < NG)
            def _():
                we_copy(g + _DEPTH, slot).start()

            @pl.when((r == B - 1) & (p + 2 < NP))
            def _():
                pos_copy(p + 2, pslot).start()

            return None

        jax.lax.fori_loop(0, NG, step, None)

        for k in range(_DEPTH):
            out_copy(NG - _DEPTH + k, k).wait()

    return body


def kernel(word_embeddings, pos_table):
    B, S, D = word_embeddings.shape
    we2 = word_embeddings.reshape(B * S, D)
    NG = (B * S) // _CHUNK
    NP = S // _CHUNK
    out = pl.pallas_call(
        _make_body(B, S, D, NG, NP),
        in_specs=[
            pl.BlockSpec(memory_space=pl.ANY),
            pl.BlockSpec(memory_space=pl.ANY),
        ],
        out_specs=pl.BlockSpec(memory_space=pl.ANY),
        out_shape=jax.ShapeDtypeStruct((B * S, D), word_embeddings.dtype),
        scratch_shapes=[
            pltpu.VMEM((_DEPTH, _CHUNK, D), word_embeddings.dtype),
            pltpu.VMEM((2, _CHUNK, D), word_embeddings.dtype),
            pltpu.VMEM((_DEPTH, _CHUNK, D), word_embeddings.dtype),
            pltpu.SemaphoreType.DMA((_DEPTH,)),
            pltpu.SemaphoreType.DMA((2,)),
            pltpu.SemaphoreType.DMA((_DEPTH,)),
        ],
    )(we2, pos_table)
    return out.reshape(B, S, D)


# speedup vs baseline: 1.0541x; 1.0086x over previous
import jax
import jax.numpy as jnp
from jax.experimental import pallas as pl
from jax.experimental.pallas import tpu as pltpu

_CHUNK = 1024
_DEPTH = 6


def _row_start(g, B, S):
    p = g // B
    r = g % B
    return r * S + p * _CHUNK


def _make_body(B, S, D, NG, NP):
    def body(we_hbm, pos_hbm, o_hbm, we_buf, pos_buf, out_buf,
             we_sem, pos_sem, out_sem):
        def we_copy(g, slot):
            rs = _row_start(g, B, S)
            return pltpu.make_async_copy(
                we_hbm.at[pl.ds(rs, _CHUNK), :], we_buf.at[slot],
                we_sem.at[slot])

        def pos_copy(p, pslot):
            return pltpu.make_async_copy(
                pos_hbm.at[pl.ds(p * _CHUNK, _CHUNK), :], pos_buf.at[pslot],
                pos_sem.at[pslot])

        def out_copy(g, slot):
            rs = _row_start(g, B, S)
            return pltpu.make_async_copy(
                out_buf.at[slot], o_hbm.at[pl.ds(rs, _CHUNK), :],
                out_sem.at[slot])

        for k in range(_DEPTH):
            we_copy(k, k).start()
        pos_copy(0, 0).start()
        pos_copy(1, 1).start()

        def step(g, _):
            slot = jax.lax.rem(g, _DEPTH)
            p = g // B
            r = jax.lax.rem(g, B)
            pslot = jax.lax.rem(p, 2)

            we_copy(g, slot).wait()

            @pl.when(r == 0)
            def _():
                pos_copy(p, pslot).wait()

            @pl.when(g >= _DEPTH)
            def _():
                out_copy(g - _DEPTH, slot).wait()

            out_buf[slot] = we_buf[slot] + pos_buf[pslot]
            out_copy(g, slot).start()

            @pl.when(g + _DEPTH < NG)
            def _():
                we_copy(g + _DEPTH, slot).start()

            @pl.when((r == B - 1) & (p + 2 < NP))
            def _():
                pos_copy(p + 2, pslot).start()

            return None

        jax.lax.fori_loop(0, NG, step, None)

        for k in range(_DEPTH):
            out_copy(NG - _DEPTH + k, k).wait()

    return body


def kernel(word_embeddings, pos_table):
    B, S, D = word_embeddings.shape
    we2 = word_embeddings.reshape(B * S, D)
    NG = (B * S) // _CHUNK
    NP = S // _CHUNK
    out = pl.pallas_call(
        _make_body(B, S, D, NG, NP),
        in_specs=[
            pl.BlockSpec(memory_space=pl.ANY),
            pl.BlockSpec(memory_space=pl.ANY),
        ],
        out_specs=pl.BlockSpec(memory_space=pl.ANY),
        out_shape=jax.ShapeDtypeStruct((B * S, D), word_embeddings.dtype),
        scratch_shapes=[
            pltpu.VMEM((_DEPTH, _CHUNK, D), word_embeddings.dtype),
            pltpu.VMEM((2, _CHUNK, D), word_embeddings.dtype),
            pltpu.VMEM((_DEPTH, _CHUNK, D), word_embeddings.dtype),
            pltpu.SemaphoreType.DMA((_DEPTH,)),
            pltpu.SemaphoreType.DMA((2,)),
            pltpu.SemaphoreType.DMA((_DEPTH,)),
        ],
    )(we2, pos_table)
    return out.reshape(B, S, D)
